# baseline (speedup 1.0000x reference)
import functools

import jax
import jax.numpy as jnp
from jax import lax
from jax.experimental import pallas as pl
from jax.experimental.pallas import tpu as pltpu
from jax.experimental.pallas import tpu_sc as plsc

_G = 8
_NBUF = 2


@functools.cache
def _make_gather(V, D, BATCH, HIST):
    info = plsc.get_sparse_core_info()
    NC, NS = info.num_cores, info.num_subcores
    NW = NC * NS
    assert BATCH % NW == 0
    rows_w = BATCH // NW
    G, nbuf = _G, _NBUF
    assert rows_w % (G * nbuf) == 0
    n_pairs = rows_w // (G * nbuf)
    mesh = plsc.VectorSubcoreMesh(core_axis_name="c", subcore_axis_name="s")

    @functools.partial(
        pl.kernel,
        mesh=mesh,
        out_type=jax.ShapeDtypeStruct((BATCH, HIST, D), jnp.float32),
        scratch_types=[
            pltpu.VMEM((rows_w, HIST), jnp.int32),
            [pltpu.VMEM((G, HIST, D), jnp.float32) for _ in range(nbuf)],
            [pltpu.SemaphoreType.DMA for _ in range(nbuf)],
            [pltpu.SemaphoreType.DMA for _ in range(nbuf)],
        ],
        compiler_params=pltpu.CompilerParams(use_tc_tiling_on_sc=False),
    )
    def gather_kernel(feat_hbm, table_hbm, out_hbm, idx_v, rbs, gsems, ssems):
        wid = lax.axis_index("s") * NC + lax.axis_index("c")
        base = wid * rows_w
        pltpu.sync_copy(feat_hbm.at[pl.ds(base, rows_w)], idx_v)

        def fire_gathers(grp, b):
            for r in range(G):
                pltpu.async_copy(
                    table_hbm.at[idx_v.at[grp * G + r]], rbs[b].at[r], gsems[b]
                )

        def drain_gathers(b):
            pltpu.make_async_copy(out_hbm.at[pl.ds(0, G)], rbs[b], gsems[b]).wait()

        def start_store(grp, b):
            pltpu.async_copy(rbs[b], out_hbm.at[pl.ds(base + grp * G, G)], ssems[b])

        def wait_store(b):
            pltpu.make_async_copy(rbs[b], out_hbm.at[pl.ds(0, G)], ssems[b]).wait()

        def run_pair(p, first):
            for b in range(nbuf):
                if not first:
                    wait_store(b)
                fire_gathers(p * nbuf + b, b)
            for b in range(nbuf):
                drain_gathers(b)
                start_store(p * nbuf + b, b)

        run_pair(0, True)

        def pair_body(p, carry):
            run_pair(p, False)
            return carry

        lax.fori_loop(1, n_pairs, pair_body, 0)
        for b in range(nbuf):
            wait_store(b)

    return gather_kernel


def kernel(feature, table):
    batch, hist = feature.shape
    dim = table.shape[1]
    return _make_gather(table.shape[0], dim, batch, hist)(feature, table)
